# Initial kernel scaffold; baseline (speedup 1.0000x reference)
import jax
import jax.numpy as jnp
from jax.experimental import pallas as pl


def kernel(x, W_router, W1, W2):
    raise NotImplementedError("write your pallas kernel here")


# speedup vs baseline: 6.4234x; 6.4234x over previous
import functools

import jax
import jax.numpy as jnp
from jax import lax
from jax.experimental import pallas as pl
from jax.experimental.pallas import tpu as pltpu
from jax.experimental.pallas import tpu_sc as plsc

DIM = 1024
FF = 4096
E = 8
K = 2
BLK = 256

NC = 2
NS = 16
NW = NC * NS
LANES = 16


def _cumsum_rows(a, n):
    s = 1
    while s < n:
        z = jnp.zeros(a.shape[:1] + (s,), a.dtype)
        a = a + jnp.concatenate([z, a[:, :-s]], axis=1)
        s *= 2
    return a


def _cumsum_cols(a, n):
    s = 1
    while s < n:
        z = jnp.zeros((s,) + a.shape[1:], a.dtype)
        a = a + jnp.concatenate([z, a[:-s]], axis=0)
        s *= 2
    return a


def _router_body(N, P, NB, x_ref, wr_ref, dst_ref, w_ref, be_ref):
    xf = x_ref[...]
    wr = wr_ref[...]
    logits = lax.dot_general(
        wr, xf, (((1,), (1,)), ((), ())),
        preferred_element_type=jnp.float32)
    eio = lax.broadcasted_iota(jnp.int32, (E, N), 0)
    neg = jnp.float32(-1e30)

    m1 = jnp.max(logits, axis=0, keepdims=True)
    oh1 = (logits == m1).astype(jnp.int32)
    first1 = (_cumsum_cols(oh1, E) == 1) & (oh1 == 1)
    i1 = jnp.sum(jnp.where(first1, eio, 0), axis=0, keepdims=True)

    masked = jnp.where(first1, neg, logits)
    m2 = jnp.max(masked, axis=0, keepdims=True)
    oh2 = (masked == m2).astype(jnp.int32)
    first2 = (_cumsum_cols(oh2, E) == 1) & (oh2 == 1)
    i2 = jnp.sum(jnp.where(first2, eio, 0), axis=0, keepdims=True)

    e2 = jnp.exp(m2 - m1)
    w1 = 1.0 / (1.0 + e2)
    w2 = e2 / (1.0 + e2)

    idx_cat = jnp.concatenate([i1, i2], axis=1)
    w_cat = jnp.concatenate([w1, w2], axis=1)

    onehot = (lax.broadcasted_iota(jnp.int32, (E, P), 0) ==
              jnp.broadcast_to(idx_cat, (E, P))).astype(jnp.int32)
    ranks_incl = _cumsum_rows(onehot, P)
    counts = ranks_incl[:, P - 1:P]
    rank = jnp.sum((ranks_incl - onehot) * onehot, axis=0, keepdims=True)

    blocks_e = (counts + (BLK - 1)) // BLK
    bstart = _cumsum_cols(blocks_e, E) - blocks_e
    pad_off = bstart * BLK
    dst = jnp.sum(pad_off * onehot, axis=0, keepdims=True) + rank

    biota = lax.broadcasted_iota(jnp.int32, (E, NB), 1)
    inblk = (biota >= bstart) & (biota < bstart + blocks_e)
    eio_b = lax.broadcasted_iota(jnp.int32, (E, NB), 0)
    be = jnp.max(jnp.where(inblk, eio_b, 0), axis=0, keepdims=True)

    dst_ref[...] = dst
    w_ref[...] = w_cat
    be_ref[...] = be


def _ffn_body(be_ref, x_ref, w1_ref, w2_ref, o_ref):
    j = pl.program_id(1)
    xb = x_ref[...]
    h = lax.dot_general(
        xb, w1_ref[0], (((1,), (0,)), ((), ())),
        preferred_element_type=jnp.float32)
    h = 0.5 * h * (1.0 + lax.erf(h * 0.7071067811865476))
    y = lax.dot_general(
        h, w2_ref[0], (((1,), (0,)), ((), ())),
        preferred_element_type=jnp.float32)

    @pl.when(j == 0)
    def _():
        o_ref[...] = y

    @pl.when(j != 0)
    def _():
        o_ref[...] += y


def _dispatch_body(N, P, xf_hbm, dst_hbm, xs_hbm, idx_v, rows_v, sem):
    wid = lax.axis_index("s") * NC + lax.axis_index("c")
    pairs_per_w = P // NW
    chunk = 64
    base = wid * pairs_per_w
    for c in range(pairs_per_w // chunk):
        off = base + c * chunk
        pltpu.sync_copy(dst_hbm.at[pl.ds(off, chunk)], idx_v)
        tok0 = lax.rem(off, N)
        pltpu.sync_copy(xf_hbm.at[pl.ds(tok0, chunk)], rows_v)
        pltpu.async_copy(rows_v, xs_hbm.at[idx_v], sem).wait()


def _combine_body(N, P, y_hbm, dst_hbm, wb_hbm, out_hbm,
                  i1_v, i2_v, r1_v, r2_v, w1_v, w2_v, o_v, sem):
    wid = lax.axis_index("s") * NC + lax.axis_index("c")
    tok_per_w = N // NW
    chunk = 32
    base = wid * tok_per_w
    for c in range(tok_per_w // chunk):
        t0 = base + c * chunk
        pltpu.sync_copy(dst_hbm.at[pl.ds(t0, chunk)], i1_v)
        pltpu.sync_copy(dst_hbm.at[pl.ds(N + t0, chunk)], i2_v)
        pltpu.sync_copy(wb_hbm.at[pl.ds(t0, chunk)], w1_v)
        pltpu.sync_copy(wb_hbm.at[pl.ds(N + t0, chunk)], w2_v)
        pltpu.async_copy(y_hbm.at[i1_v], r1_v, sem).wait()
        pltpu.async_copy(y_hbm.at[i2_v], r2_v, sem).wait()

        def row(i, _):
            wa = w1_v[i, :]
            wb = w2_v[i, :]
            def col(j, _):
                a = r1_v[i, pl.ds(j * LANES, LANES)]
                b = r2_v[i, pl.ds(j * LANES, LANES)]
                o_v[i, pl.ds(j * LANES, LANES)] = wa * a + wb * b
                return 0
            return lax.fori_loop(0, DIM // LANES, col, 0)
        lax.fori_loop(0, chunk, row, 0)
        pltpu.sync_copy(o_v, out_hbm.at[pl.ds(t0, chunk)])


def kernel(x, W_router, W1, W2):
    B, T, _ = x.shape
    N = B * T
    P = N * K
    NB = P // BLK + E
    PAD = NB * BLK
    xf = x.reshape(N, DIM)

    dst2, w2d, be2 = pl.pallas_call(
        functools.partial(_router_body, N, P, NB),
        out_shape=(
            jax.ShapeDtypeStruct((1, P), jnp.int32),
            jax.ShapeDtypeStruct((1, P), jnp.float32),
            jax.ShapeDtypeStruct((1, NB), jnp.int32),
        ),
    )(xf, W_router)
    dst = dst2.reshape(P)
    w_flat = w2d.reshape(P)
    block_expert = be2.reshape(NB)

    mesh = plsc.VectorSubcoreMesh(core_axis_name="c", subcore_axis_name="s",
                                  num_cores=NC, num_subcores=NS)
    x_sorted = pl.kernel(
        functools.partial(_dispatch_body, N, P),
        out_type=jax.ShapeDtypeStruct((PAD, DIM), jnp.float32),
        mesh=mesh,
        scratch_types=[
            pltpu.VMEM((64,), jnp.int32),
            pltpu.VMEM((64, DIM), jnp.float32),
            pltpu.SemaphoreType.DMA,
        ],
    )(xf, dst)

    NFT = 2
    FT = FF // NFT
    y_sorted = pl.pallas_call(
        _ffn_body,
        grid_spec=pltpu.PrefetchScalarGridSpec(
            num_scalar_prefetch=1,
            grid=(NB, NFT),
            in_specs=[
                pl.BlockSpec((BLK, DIM), lambda b, j, be: (b, 0)),
                pl.BlockSpec((1, DIM, FT), lambda b, j, be: (be[b], 0, j)),
                pl.BlockSpec((1, FT, DIM), lambda b, j, be: (be[b], j, 0)),
            ],
            out_specs=pl.BlockSpec((BLK, DIM), lambda b, j, be: (b, 0)),
        ),
        out_shape=jax.ShapeDtypeStruct((PAD, DIM), jnp.float32),
    )(block_expert, x_sorted, W1, W2)

    wb = jnp.broadcast_to(w_flat[:, None], (P, LANES))
    out = pl.kernel(
        functools.partial(_combine_body, N, P),
        out_type=jax.ShapeDtypeStruct((N, DIM), jnp.float32),
        mesh=mesh,
        scratch_types=[
            pltpu.VMEM((32,), jnp.int32),
            pltpu.VMEM((32,), jnp.int32),
            pltpu.VMEM((32, DIM), jnp.float32),
            pltpu.VMEM((32, DIM), jnp.float32),
            pltpu.VMEM((32, LANES), jnp.float32),
            pltpu.VMEM((32, LANES), jnp.float32),
            pltpu.VMEM((32, DIM), jnp.float32),
            pltpu.SemaphoreType.DMA,
        ],
    )(y_sorted, dst, wb)

    return out.reshape(B, T, DIM)


# speedup vs baseline: 7.8648x; 1.2244x over previous
import functools

import jax
import jax.numpy as jnp
from jax import lax
from jax.experimental import pallas as pl
from jax.experimental.pallas import tpu as pltpu
from jax.experimental.pallas import tpu_sc as plsc

DIM = 1024
FF = 4096
E = 8
K = 2
BLK = 256

NC = 2
NS = 16
NW = NC * NS
LANES = 16


def _cumsum_rows(a, n):
    s = 1
    while s < n:
        z = jnp.zeros(a.shape[:1] + (s,), a.dtype)
        a = a + jnp.concatenate([z, a[:, :-s]], axis=1)
        s *= 2
    return a


def _cumsum_cols(a, n):
    s = 1
    while s < n:
        z = jnp.zeros((s,) + a.shape[1:], a.dtype)
        a = a + jnp.concatenate([z, a[:-s]], axis=0)
        s *= 2
    return a


def _router_body(N, P, NB, x_ref, wr_ref, dst_ref, w_ref, be_ref):
    xf = x_ref[...]
    wr = wr_ref[...]
    logits = lax.dot_general(
        wr, xf, (((1,), (1,)), ((), ())),
        preferred_element_type=jnp.float32)
    eio = lax.broadcasted_iota(jnp.int32, (E, N), 0)
    neg = jnp.float32(-1e30)

    m1 = jnp.max(logits, axis=0, keepdims=True)
    oh1 = (logits == m1).astype(jnp.int32)
    first1 = (_cumsum_cols(oh1, E) == 1) & (oh1 == 1)
    i1 = jnp.sum(jnp.where(first1, eio, 0), axis=0, keepdims=True)

    masked = jnp.where(first1, neg, logits)
    m2 = jnp.max(masked, axis=0, keepdims=True)
    oh2 = (masked == m2).astype(jnp.int32)
    first2 = (_cumsum_cols(oh2, E) == 1) & (oh2 == 1)
    i2 = jnp.sum(jnp.where(first2, eio, 0), axis=0, keepdims=True)

    e2 = jnp.exp(m2 - m1)
    w1 = 1.0 / (1.0 + e2)
    w2 = e2 / (1.0 + e2)

    idx_cat = jnp.concatenate([i1, i2], axis=1)
    w_cat = jnp.concatenate([w1, w2], axis=1)

    onehot = (lax.broadcasted_iota(jnp.int32, (E, P), 0) ==
              jnp.broadcast_to(idx_cat, (E, P))).astype(jnp.int32)
    ranks_incl = _cumsum_rows(onehot, P)
    counts = ranks_incl[:, P - 1:P]
    rank = jnp.sum((ranks_incl - onehot) * onehot, axis=0, keepdims=True)

    blocks_e = (counts + (BLK - 1)) // BLK
    bstart = _cumsum_cols(blocks_e, E) - blocks_e
    pad_off = bstart * BLK
    dst = jnp.sum(pad_off * onehot, axis=0, keepdims=True) + rank

    biota = lax.broadcasted_iota(jnp.int32, (E, NB), 1)
    inblk = (biota >= bstart) & (biota < bstart + blocks_e)
    eio_b = lax.broadcasted_iota(jnp.int32, (E, NB), 0)
    be = jnp.max(jnp.where(inblk, eio_b, 0), axis=0, keepdims=True)

    dst_ref[...] = dst
    w_ref[...] = w_cat
    be_ref[...] = be


def _ffn_body(be_ref, x_ref, w1_ref, w2_ref, o_ref):
    xb = x_ref[...].astype(jnp.bfloat16)
    h = lax.dot_general(
        xb, w1_ref[0], (((1,), (0,)), ((), ())),
        preferred_element_type=jnp.float32)
    h = 0.5 * h * (1.0 + lax.erf(h * 0.7071067811865476))
    o_ref[...] = lax.dot_general(
        h.astype(jnp.bfloat16), w2_ref[0], (((1,), (0,)), ((), ())),
        preferred_element_type=jnp.float32)


def _dispatch_body(N, P, xf_hbm, dst_hbm, xs_hbm, idx_v, rows_v, sem):
    wid = lax.axis_index("s") * NC + lax.axis_index("c")
    pairs_per_w = P // NW
    chunk = 64
    base = wid * pairs_per_w
    for c in range(pairs_per_w // chunk):
        off = base + c * chunk
        pltpu.sync_copy(dst_hbm.at[pl.ds(off, chunk)], idx_v)
        tok0 = lax.rem(off, N)
        pltpu.sync_copy(xf_hbm.at[pl.ds(tok0, chunk)], rows_v)
        pltpu.async_copy(rows_v, xs_hbm.at[idx_v], sem).wait()


def _combine_body(N, P, y_hbm, dst_hbm, wb_hbm, out_hbm,
                  i1_v, i2_v, r1_v, r2_v, w1_v, w2_v, o_v, sem):
    wid = lax.axis_index("s") * NC + lax.axis_index("c")
    tok_per_w = N // NW
    chunk = 32
    base = wid * tok_per_w
    for c in range(tok_per_w // chunk):
        t0 = base + c * chunk
        pltpu.sync_copy(dst_hbm.at[pl.ds(t0, chunk)], i1_v)
        pltpu.sync_copy(dst_hbm.at[pl.ds(N + t0, chunk)], i2_v)
        pltpu.sync_copy(wb_hbm.at[pl.ds(t0, chunk)], w1_v)
        pltpu.sync_copy(wb_hbm.at[pl.ds(N + t0, chunk)], w2_v)
        pltpu.async_copy(y_hbm.at[i1_v], r1_v, sem).wait()
        pltpu.async_copy(y_hbm.at[i2_v], r2_v, sem).wait()

        def row(i, _):
            wa = w1_v[i, :]
            wb = w2_v[i, :]
            def col(j, _):
                a = r1_v[i, pl.ds(j * LANES, LANES)]
                b = r2_v[i, pl.ds(j * LANES, LANES)]
                o_v[i, pl.ds(j * LANES, LANES)] = wa * a + wb * b
                return 0
            return lax.fori_loop(0, DIM // LANES, col, 0)
        lax.fori_loop(0, chunk, row, 0)
        pltpu.sync_copy(o_v, out_hbm.at[pl.ds(t0, chunk)])


def kernel(x, W_router, W1, W2):
    B, T, _ = x.shape
    N = B * T
    P = N * K
    NB = P // BLK + E
    PAD = NB * BLK
    xf = x.reshape(N, DIM)

    dst2, w2d, be2 = pl.pallas_call(
        functools.partial(_router_body, N, P, NB),
        out_shape=(
            jax.ShapeDtypeStruct((1, P), jnp.int32),
            jax.ShapeDtypeStruct((1, P), jnp.float32),
            jax.ShapeDtypeStruct((1, NB), jnp.int32),
        ),
    )(xf, W_router)
    dst = dst2.reshape(P)
    w_flat = w2d.reshape(P)
    block_expert = be2.reshape(NB)

    mesh = plsc.VectorSubcoreMesh(core_axis_name="c", subcore_axis_name="s",
                                  num_cores=NC, num_subcores=NS)
    x_sorted = pl.kernel(
        functools.partial(_dispatch_body, N, P),
        out_type=jax.ShapeDtypeStruct((PAD, DIM), jnp.float32),
        mesh=mesh,
        scratch_types=[
            pltpu.VMEM((64,), jnp.int32),
            pltpu.VMEM((64, DIM), jnp.float32),
            pltpu.SemaphoreType.DMA,
        ],
    )(xf, dst)

    W1b = W1.astype(jnp.bfloat16)
    W2b = W2.astype(jnp.bfloat16)
    y_sorted = pl.pallas_call(
        _ffn_body,
        grid_spec=pltpu.PrefetchScalarGridSpec(
            num_scalar_prefetch=1,
            grid=(NB,),
            in_specs=[
                pl.BlockSpec((BLK, DIM), lambda b, be: (b, 0)),
                pl.BlockSpec((1, DIM, FF), lambda b, be: (be[b], 0, 0)),
                pl.BlockSpec((1, FF, DIM), lambda b, be: (be[b], 0, 0)),
            ],
            out_specs=pl.BlockSpec((BLK, DIM), lambda b, be: (b, 0)),
        ),
        out_shape=jax.ShapeDtypeStruct((PAD, DIM), jnp.float32),
    )(block_expert, x_sorted, W1b, W2b)

    wb = jnp.broadcast_to(w_flat[:, None], (P, LANES))
    out = pl.kernel(
        functools.partial(_combine_body, N, P),
        out_type=jax.ShapeDtypeStruct((N, DIM), jnp.float32),
        mesh=mesh,
        scratch_types=[
            pltpu.VMEM((32,), jnp.int32),
            pltpu.VMEM((32,), jnp.int32),
            pltpu.VMEM((32, DIM), jnp.float32),
            pltpu.VMEM((32, DIM), jnp.float32),
            pltpu.VMEM((32, LANES), jnp.float32),
            pltpu.VMEM((32, LANES), jnp.float32),
            pltpu.VMEM((32, DIM), jnp.float32),
            pltpu.SemaphoreType.DMA,
        ],
    )(y_sorted, dst, wb)

    return out.reshape(B, T, DIM)


# speedup vs baseline: 9.4708x; 1.2042x over previous
import functools

import jax
import jax.numpy as jnp
from jax import lax
from jax.experimental import pallas as pl
from jax.experimental.pallas import tpu as pltpu
from jax.experimental.pallas import tpu_sc as plsc

DIM = 1024
FF = 4096
E = 8
K = 2
BLK = 256

NC = 2
NS = 16
NW = NC * NS
LANES = 16


def _cumsum_rows(a, n):
    s = 1
    while s < n:
        z = jnp.zeros(a.shape[:1] + (s,), a.dtype)
        a = a + jnp.concatenate([z, a[:, :-s]], axis=1)
        s *= 2
    return a


def _cumsum_cols(a, n):
    s = 1
    while s < n:
        z = jnp.zeros((s,) + a.shape[1:], a.dtype)
        a = a + jnp.concatenate([z, a[:-s]], axis=0)
        s *= 2
    return a


def _router_body(N, P, NB, x_ref, wr_ref, dst_ref, w_ref, be_ref):
    xf = x_ref[...]
    wr = wr_ref[...]
    logits = lax.dot_general(
        wr, xf, (((1,), (1,)), ((), ())),
        preferred_element_type=jnp.float32)
    eio = lax.broadcasted_iota(jnp.int32, (E, N), 0)
    neg = jnp.float32(-1e30)

    m1 = jnp.max(logits, axis=0, keepdims=True)
    oh1 = (logits == m1).astype(jnp.int32)
    first1 = (_cumsum_cols(oh1, E) == 1) & (oh1 == 1)
    i1 = jnp.sum(jnp.where(first1, eio, 0), axis=0, keepdims=True)

    masked = jnp.where(first1, neg, logits)
    m2 = jnp.max(masked, axis=0, keepdims=True)
    oh2 = (masked == m2).astype(jnp.int32)
    first2 = (_cumsum_cols(oh2, E) == 1) & (oh2 == 1)
    i2 = jnp.sum(jnp.where(first2, eio, 0), axis=0, keepdims=True)

    e2 = jnp.exp(m2 - m1)
    w1 = 1.0 / (1.0 + e2)
    w2 = e2 / (1.0 + e2)

    idx_cat = jnp.concatenate([i1, i2], axis=1)
    w_cat = jnp.concatenate([w1, w2], axis=1)

    onehot = (lax.broadcasted_iota(jnp.int32, (E, P), 0) ==
              jnp.broadcast_to(idx_cat, (E, P))).astype(jnp.int32)
    ranks_incl = _cumsum_rows(onehot, P)
    counts = ranks_incl[:, P - 1:P]
    rank = jnp.sum((ranks_incl - onehot) * onehot, axis=0, keepdims=True)

    blocks_e = (counts + (BLK - 1)) // BLK
    bstart = _cumsum_cols(blocks_e, E) - blocks_e
    pad_off = bstart * BLK
    dst = jnp.sum(pad_off * onehot, axis=0, keepdims=True) + rank

    biota = lax.broadcasted_iota(jnp.int32, (E, NB), 1)
    inblk = (biota >= bstart) & (biota < bstart + blocks_e)
    eio_b = lax.broadcasted_iota(jnp.int32, (E, NB), 0)
    be = jnp.max(jnp.where(inblk, eio_b, 0), axis=0, keepdims=True)

    dst_ref[...] = dst
    w_ref[...] = w_cat
    be_ref[...] = be


def _ffn_body(be_ref, x_ref, w1_ref, w2_ref, o_ref):
    xb = x_ref[...]
    h = lax.dot_general(
        xb, w1_ref[0], (((1,), (0,)), ((), ())),
        preferred_element_type=jnp.float32)
    h = 0.5 * h * (1.0 + lax.erf(h * 0.7071067811865476))
    o_ref[...] = lax.dot_general(
        h, w2_ref[0], (((1,), (0,)), ((), ())),
        preferred_element_type=jnp.float32)


def _dispatch_body(N, P, xf_hbm, dst_hbm, xs_hbm, idx_v, rows_v, sem):
    wid = lax.axis_index("s") * NC + lax.axis_index("c")
    pairs_per_w = P // NW
    chunk = 64
    base = wid * pairs_per_w
    for c in range(pairs_per_w // chunk):
        off = base + c * chunk
        pltpu.sync_copy(dst_hbm.at[pl.ds(off, chunk)], idx_v)
        tok0 = lax.rem(off, N)
        pltpu.sync_copy(xf_hbm.at[pl.ds(tok0, chunk)], rows_v)
        pltpu.async_copy(rows_v, xs_hbm.at[idx_v], sem).wait()


def _combine_body(N, P, y_hbm, dst_hbm, wb_hbm, out_hbm,
                  i1_v, i2_v, r1_v, r2_v, w1_v, w2_v, o_v, sem):
    wid = lax.axis_index("s") * NC + lax.axis_index("c")
    tok_per_w = N // NW
    chunk = 32
    base = wid * tok_per_w
    for c in range(tok_per_w // chunk):
        t0 = base + c * chunk
        pltpu.sync_copy(dst_hbm.at[pl.ds(t0, chunk)], i1_v)
        pltpu.sync_copy(dst_hbm.at[pl.ds(N + t0, chunk)], i2_v)
        pltpu.sync_copy(wb_hbm.at[pl.ds(t0, chunk)], w1_v)
        pltpu.sync_copy(wb_hbm.at[pl.ds(N + t0, chunk)], w2_v)
        pltpu.async_copy(y_hbm.at[i1_v], r1_v, sem).wait()
        pltpu.async_copy(y_hbm.at[i2_v], r2_v, sem).wait()

        def row(i, _):
            wa = w1_v[i, :]
            wb = w2_v[i, :]
            def col(j, _):
                a = r1_v[i, pl.ds(j * LANES, LANES)]
                b = r2_v[i, pl.ds(j * LANES, LANES)]
                o_v[i, pl.ds(j * LANES, LANES)] = wa * a + wb * b
                return 0
            return lax.fori_loop(0, DIM // LANES, col, 0)
        lax.fori_loop(0, chunk, row, 0)
        pltpu.sync_copy(o_v, out_hbm.at[pl.ds(t0, chunk)])


def kernel(x, W_router, W1, W2):
    B, T, _ = x.shape
    N = B * T
    P = N * K
    NB = P // BLK + E
    PAD = NB * BLK
    xf = x.reshape(N, DIM)

    dst2, w2d, be2 = pl.pallas_call(
        functools.partial(_router_body, N, P, NB),
        out_shape=(
            jax.ShapeDtypeStruct((1, P), jnp.int32),
            jax.ShapeDtypeStruct((1, P), jnp.float32),
            jax.ShapeDtypeStruct((1, NB), jnp.int32),
        ),
    )(xf, W_router)
    dst = dst2.reshape(P)
    w_flat = w2d.reshape(P)
    block_expert = be2.reshape(NB)

    mesh = plsc.VectorSubcoreMesh(core_axis_name="c", subcore_axis_name="s",
                                  num_cores=NC, num_subcores=NS)
    x_sorted = pl.kernel(
        functools.partial(_dispatch_body, N, P),
        out_type=jax.ShapeDtypeStruct((PAD, DIM), jnp.float32),
        mesh=mesh,
        scratch_types=[
            pltpu.VMEM((64,), jnp.int32),
            pltpu.VMEM((64, DIM), jnp.float32),
            pltpu.SemaphoreType.DMA,
        ],
    )(xf, dst)

    y_sorted = pl.pallas_call(
        _ffn_body,
        grid_spec=pltpu.PrefetchScalarGridSpec(
            num_scalar_prefetch=1,
            grid=(NB,),
            in_specs=[
                pl.BlockSpec((BLK, DIM), lambda b, be: (b, 0)),
                pl.BlockSpec((1, DIM, FF), lambda b, be: (be[b], 0, 0),
                             pipeline_mode=pl.Buffered(buffer_count=2)),
                pl.BlockSpec((1, FF, DIM), lambda b, be: (be[b], 0, 0),
                             pipeline_mode=pl.Buffered(buffer_count=1)),
            ],
            out_specs=pl.BlockSpec((BLK, DIM), lambda b, be: (b, 0)),
        ),
        out_shape=jax.ShapeDtypeStruct((PAD, DIM), jnp.float32),
    )(block_expert, x_sorted, W1, W2)

    wb = jnp.broadcast_to(w_flat[:, None], (P, LANES))
    out = pl.kernel(
        functools.partial(_combine_body, N, P),
        out_type=jax.ShapeDtypeStruct((N, DIM), jnp.float32),
        mesh=mesh,
        scratch_types=[
            pltpu.VMEM((32,), jnp.int32),
            pltpu.VMEM((32,), jnp.int32),
            pltpu.VMEM((32, DIM), jnp.float32),
            pltpu.VMEM((32, DIM), jnp.float32),
            pltpu.VMEM((32, LANES), jnp.float32),
            pltpu.VMEM((32, LANES), jnp.float32),
            pltpu.VMEM((32, DIM), jnp.float32),
            pltpu.SemaphoreType.DMA,
        ],
    )(y_sorted, dst, wb)

    return out.reshape(B, T, DIM)


# speedup vs baseline: 10.4227x; 1.1005x over previous
import functools

import jax
import jax.numpy as jnp
from jax import lax
from jax.experimental import pallas as pl
from jax.experimental.pallas import tpu as pltpu
from jax.experimental.pallas import tpu_sc as plsc

DIM = 1024
FF = 4096
E = 8
K = 2
BLK = 256

NC = 2
NS = 16
NW = NC * NS
LANES = 16


def _cumsum_rows(a, n):
    s = 1
    while s < n:
        z = jnp.zeros(a.shape[:1] + (s,), a.dtype)
        a = a + jnp.concatenate([z, a[:, :-s]], axis=1)
        s *= 2
    return a


def _cumsum_cols(a, n):
    s = 1
    while s < n:
        z = jnp.zeros((s,) + a.shape[1:], a.dtype)
        a = a + jnp.concatenate([z, a[:-s]], axis=0)
        s *= 2
    return a


def _router_body(N, P, NB, x_ref, wr_ref, dst_ref, w_ref, be_ref):
    xf = x_ref[...]
    wr = wr_ref[...]
    logits = lax.dot_general(
        wr, xf, (((1,), (1,)), ((), ())),
        preferred_element_type=jnp.float32)
    eio = lax.broadcasted_iota(jnp.int32, (E, N), 0)
    neg = jnp.float32(-1e30)

    m1 = jnp.max(logits, axis=0, keepdims=True)
    oh1 = (logits == m1).astype(jnp.int32)
    first1 = (_cumsum_cols(oh1, E) == 1) & (oh1 == 1)
    i1 = jnp.sum(jnp.where(first1, eio, 0), axis=0, keepdims=True)

    masked = jnp.where(first1, neg, logits)
    m2 = jnp.max(masked, axis=0, keepdims=True)
    oh2 = (masked == m2).astype(jnp.int32)
    first2 = (_cumsum_cols(oh2, E) == 1) & (oh2 == 1)
    i2 = jnp.sum(jnp.where(first2, eio, 0), axis=0, keepdims=True)

    e2 = jnp.exp(m2 - m1)
    w1 = 1.0 / (1.0 + e2)
    w2 = e2 / (1.0 + e2)

    idx_cat = jnp.concatenate([i1, i2], axis=1)
    w_cat = jnp.concatenate([w1, w2], axis=1)

    onehot = (lax.broadcasted_iota(jnp.int32, (E, P), 0) ==
              jnp.broadcast_to(idx_cat, (E, P))).astype(jnp.int32)
    ranks_incl = _cumsum_rows(onehot, P)
    counts = ranks_incl[:, P - 1:P]
    rank = jnp.sum((ranks_incl - onehot) * onehot, axis=0, keepdims=True)

    blocks_e = (counts + (BLK - 1)) // BLK
    bstart = _cumsum_cols(blocks_e, E) - blocks_e
    pad_off = bstart * BLK
    dst = jnp.sum(pad_off * onehot, axis=0, keepdims=True) + rank

    biota = lax.broadcasted_iota(jnp.int32, (E, NB), 1)
    inblk = (biota >= bstart) & (biota < bstart + blocks_e)
    eio_b = lax.broadcasted_iota(jnp.int32, (E, NB), 0)
    be = jnp.max(jnp.where(inblk, eio_b, 0), axis=0, keepdims=True)

    dst_ref[...] = dst
    w_ref[...] = w_cat
    be_ref[...] = be


def _ffn_body(be_ref, x_ref, w1_ref, w2_ref, o_ref):
    xb = x_ref[...]
    h = lax.dot_general(
        xb, w1_ref[0], (((1,), (0,)), ((), ())),
        preferred_element_type=jnp.float32)
    h = 0.5 * h * (1.0 + lax.erf(h * 0.7071067811865476))
    o_ref[...] = lax.dot_general(
        h, w2_ref[0], (((1,), (0,)), ((), ())),
        preferred_element_type=jnp.float32)


def _dispatch_body(N, P, xf_hbm, dst_hbm, xs_hbm,
                   idx0_v, idx1_v, rows_v, gsems, ssems):
    wid = lax.axis_index("s") * NC + lax.axis_index("c")
    TPW = N // NW
    CH = 16
    NCH = TPW // CH
    DEPTH = len(rows_v)
    t0 = wid * TPW
    pltpu.sync_copy(dst_hbm.at[pl.ds(t0, TPW)], idx0_v)
    pltpu.sync_copy(dst_hbm.at[pl.ds(N + t0, TPW)], idx1_v)
    for c in range(min(DEPTH, NCH)):
        pltpu.async_copy(xf_hbm.at[pl.ds(t0 + c * CH, CH)], rows_v[c],
                         gsems[c])
    for c in range(NCH):
        b = c % DEPTH
        iv0 = idx0_v[pl.ds(c * CH, CH)]
        iv1 = idx1_v[pl.ds(c * CH, CH)]
        pltpu.make_async_copy(xf_hbm.at[pl.ds(t0 + c * CH, CH)], rows_v[b],
                              gsems[b]).wait()
        s0 = pltpu.async_copy(rows_v[b], xs_hbm.at[iv0], ssems[b])
        s1 = pltpu.async_copy(rows_v[b], xs_hbm.at[iv1], ssems[b])
        if c + DEPTH < NCH:
            s0.wait()
            s1.wait()
            pltpu.async_copy(xf_hbm.at[pl.ds(t0 + (c + DEPTH) * CH, CH)],
                             rows_v[b], gsems[b])
    for c in range(max(NCH - DEPTH, 0), NCH):
        b = c % DEPTH
        iv0 = idx0_v[pl.ds(c * CH, CH)]
        iv1 = idx1_v[pl.ds(c * CH, CH)]
        pltpu.make_async_copy(rows_v[b], xs_hbm.at[iv0], ssems[b]).wait()
        pltpu.make_async_copy(rows_v[b], xs_hbm.at[iv1], ssems[b]).wait()


def _bcast_lane(vec, i):
    idx = jnp.full((LANES, 1), i, jnp.int32)
    dn = lax.GatherDimensionNumbers(
        offset_dims=(), collapsed_slice_dims=(0,), start_index_map=(0,))
    return lax.gather(vec, idx, dn, (1,),
                      mode=lax.GatherScatterMode.PROMISE_IN_BOUNDS)


def _combine_body(N, P, y_hbm, dst_hbm, wb_hbm, out_hbm,
                  i0_v, i1_v, w0_v, w1_v, r0_v, r1_v, o_v,
                  g0sems, g1sems, osems):
    wid = lax.axis_index("s") * NC + lax.axis_index("c")
    TPW = N // NW
    CH = 16
    NCH = TPW // CH
    DEPTH = len(o_v)
    t0 = wid * TPW
    pltpu.sync_copy(dst_hbm.at[pl.ds(t0, TPW)], i0_v)
    pltpu.sync_copy(dst_hbm.at[pl.ds(N + t0, TPW)], i1_v)
    pltpu.sync_copy(wb_hbm.at[pl.ds(t0, TPW)], w0_v)
    pltpu.sync_copy(wb_hbm.at[pl.ds(N + t0, TPW)], w1_v)
    for c in range(min(DEPTH, NCH)):
        pltpu.async_copy(y_hbm.at[i0_v[pl.ds(c * CH, CH)]], r0_v[c],
                         g0sems[c])
        pltpu.async_copy(y_hbm.at[i1_v[pl.ds(c * CH, CH)]], r1_v[c],
                         g1sems[c])
    for c in range(NCH):
        b = c % DEPTH
        pltpu.make_async_copy(y_hbm.at[i0_v[pl.ds(c * CH, CH)]], r0_v[b],
                              g0sems[b]).wait()
        pltpu.make_async_copy(y_hbm.at[i1_v[pl.ds(c * CH, CH)]], r1_v[b],
                              g1sems[b]).wait()
        if c >= DEPTH:
            pltpu.make_async_copy(
                o_v[b], out_hbm.at[pl.ds(t0 + (c - DEPTH) * CH, CH)],
                osems[b]).wait()

        wv0 = w0_v[pl.ds(c * CH, CH)]
        wv1 = w1_v[pl.ds(c * CH, CH)]

        def row(i, _):
            wa = _bcast_lane(wv0, i)
            wb = _bcast_lane(wv1, i)
            def col(j, _):
                a = r0_v[b][i, pl.ds(j * LANES, LANES)]
                d = r1_v[b][i, pl.ds(j * LANES, LANES)]
                o_v[b][i, pl.ds(j * LANES, LANES)] = wa * a + wb * d
                return 0
            return lax.fori_loop(0, DIM // LANES, col, 0)
        lax.fori_loop(0, CH, row, 0)
        pltpu.async_copy(o_v[b], out_hbm.at[pl.ds(t0 + c * CH, CH)], osems[b])
        if c + DEPTH < NCH:
            pltpu.async_copy(y_hbm.at[i0_v[pl.ds((c + DEPTH) * CH, CH)]],
                             r0_v[b], g0sems[b])
            pltpu.async_copy(y_hbm.at[i1_v[pl.ds((c + DEPTH) * CH, CH)]],
                             r1_v[b], g1sems[b])
    for c in range(max(NCH - DEPTH, 0), NCH):
        b = c % DEPTH
        pltpu.make_async_copy(o_v[b], out_hbm.at[pl.ds(t0 + c * CH, CH)],
                              osems[b]).wait()


def kernel(x, W_router, W1, W2):
    B, T, _ = x.shape
    N = B * T
    P = N * K
    NB = P // BLK + E
    PAD = NB * BLK
    xf = x.reshape(N, DIM)

    dst2, w2d, be2 = pl.pallas_call(
        functools.partial(_router_body, N, P, NB),
        out_shape=(
            jax.ShapeDtypeStruct((1, P), jnp.int32),
            jax.ShapeDtypeStruct((1, P), jnp.float32),
            jax.ShapeDtypeStruct((1, NB), jnp.int32),
        ),
    )(xf, W_router)
    dst = dst2.reshape(P)
    w_flat = w2d.reshape(P)
    block_expert = be2.reshape(NB)

    mesh = plsc.VectorSubcoreMesh(core_axis_name="c", subcore_axis_name="s",
                                  num_cores=NC, num_subcores=NS)
    x_sorted = pl.kernel(
        functools.partial(_dispatch_body, N, P),
        out_type=jax.ShapeDtypeStruct((PAD, DIM), jnp.float32),
        mesh=mesh,
        scratch_types=[
            pltpu.VMEM((N // NW,), jnp.int32),
            pltpu.VMEM((N // NW,), jnp.int32),
            (pltpu.VMEM((16, DIM), jnp.float32),) * 3,
            (pltpu.SemaphoreType.DMA,) * 3,
            (pltpu.SemaphoreType.DMA,) * 3,
        ],
    )(xf, dst)

    y_sorted = pl.pallas_call(
        _ffn_body,
        grid_spec=pltpu.PrefetchScalarGridSpec(
            num_scalar_prefetch=1,
            grid=(NB,),
            in_specs=[
                pl.BlockSpec((BLK, DIM), lambda b, be: (b, 0)),
                pl.BlockSpec((1, DIM, FF), lambda b, be: (be[b], 0, 0),
                             pipeline_mode=pl.Buffered(buffer_count=2)),
                pl.BlockSpec((1, FF, DIM), lambda b, be: (be[b], 0, 0),
                             pipeline_mode=pl.Buffered(buffer_count=1)),
            ],
            out_specs=pl.BlockSpec((BLK, DIM), lambda b, be: (b, 0)),
        ),
        out_shape=jax.ShapeDtypeStruct((PAD, DIM), jnp.float32),
    )(block_expert, x_sorted, W1, W2)

    TPW = N // NW
    out = pl.kernel(
        functools.partial(_combine_body, N, P),
        out_type=jax.ShapeDtypeStruct((N, DIM), jnp.float32),
        mesh=mesh,
        scratch_types=[
            pltpu.VMEM((TPW,), jnp.int32),
            pltpu.VMEM((TPW,), jnp.int32),
            pltpu.VMEM((TPW,), jnp.float32),
            pltpu.VMEM((TPW,), jnp.float32),
            (pltpu.VMEM((16, DIM), jnp.float32),) * 2,
            (pltpu.VMEM((16, DIM), jnp.float32),) * 2,
            (pltpu.VMEM((16, DIM), jnp.float32),) * 2,
            (pltpu.SemaphoreType.DMA,) * 2,
            (pltpu.SemaphoreType.DMA,) * 2,
            (pltpu.SemaphoreType.DMA,) * 2,
        ],
    )(y_sorted, dst, w_flat)

    return out.reshape(B, T, DIM)


# speedup vs baseline: 11.2821x; 1.0825x over previous
import functools

import jax
import jax.numpy as jnp
from jax import lax
from jax.experimental import pallas as pl
from jax.experimental.pallas import tpu as pltpu
from jax.experimental.pallas import tpu_sc as plsc

DIM = 1024
FF = 4096
E = 8
K = 2
BLK = 256

NC = 2
NS = 16
NW = NC * NS
LANES = 16


def _cumsum_rows(a, n):
    s = 1
    while s < n:
        z = jnp.zeros(a.shape[:1] + (s,), a.dtype)
        a = a + jnp.concatenate([z, a[:, :-s]], axis=1)
        s *= 2
    return a


def _cumsum_cols(a, n):
    s = 1
    while s < n:
        z = jnp.zeros((s,) + a.shape[1:], a.dtype)
        a = a + jnp.concatenate([z, a[:-s]], axis=0)
        s *= 2
    return a


def _router_body(N, P, NB, x_ref, wr_ref, dst_ref, w_ref, be_ref):
    xf = x_ref[...]
    wr = wr_ref[...]
    logits = lax.dot_general(
        wr, xf, (((1,), (1,)), ((), ())),
        preferred_element_type=jnp.float32)
    eio = lax.broadcasted_iota(jnp.int32, (E, N), 0)
    neg = jnp.float32(-1e30)

    m1 = jnp.max(logits, axis=0, keepdims=True)
    oh1 = (logits == m1).astype(jnp.int32)
    first1 = (_cumsum_cols(oh1, E) == 1) & (oh1 == 1)
    i1 = jnp.sum(jnp.where(first1, eio, 0), axis=0, keepdims=True)

    masked = jnp.where(first1, neg, logits)
    m2 = jnp.max(masked, axis=0, keepdims=True)
    oh2 = (masked == m2).astype(jnp.int32)
    first2 = (_cumsum_cols(oh2, E) == 1) & (oh2 == 1)
    i2 = jnp.sum(jnp.where(first2, eio, 0), axis=0, keepdims=True)

    e2 = jnp.exp(m2 - m1)
    w1 = 1.0 / (1.0 + e2)
    w2 = e2 / (1.0 + e2)

    idx_cat = jnp.concatenate([i1, i2], axis=1)
    w_cat = jnp.concatenate([w1, w2], axis=1)

    onehot = (lax.broadcasted_iota(jnp.int32, (E, P), 0) ==
              jnp.broadcast_to(idx_cat, (E, P))).astype(jnp.int32)
    ranks_incl = _cumsum_rows(onehot, P)
    counts = ranks_incl[:, P - 1:P]
    rank = jnp.sum((ranks_incl - onehot) * onehot, axis=0, keepdims=True)

    blocks_e = (counts + (BLK - 1)) // BLK
    bstart = _cumsum_cols(blocks_e, E) - blocks_e
    pad_off = bstart * BLK
    dst = jnp.sum(pad_off * onehot, axis=0, keepdims=True) + rank

    biota = lax.broadcasted_iota(jnp.int32, (E, NB), 1)
    inblk = (biota >= bstart) & (biota < bstart + blocks_e)
    eio_b = lax.broadcasted_iota(jnp.int32, (E, NB), 0)
    be = jnp.max(jnp.where(inblk, eio_b, -1), axis=0, keepdims=True)
    eio_c = lax.broadcasted_iota(jnp.int32, (E, 1), 0)
    be_last = jnp.max(jnp.where(blocks_e > 0, eio_c, 0))
    be = jnp.where(be < 0, be_last, be)
    used = jnp.sum(blocks_e).astype(jnp.int32)

    dst_ref[...] = dst
    w_ref[...] = w_cat
    be_ref[...] = jnp.concatenate(
        [be, jnp.broadcast_to(used, (1, 1))], axis=1)


def _ffn_body(NB, be_ref, x_ref, w1_ref, w2_ref, o_ref):
    @pl.when(pl.program_id(0) < be_ref[NB])
    def _():
        xb = x_ref[...]
        h = lax.dot_general(
            xb, w1_ref[0], (((1,), (0,)), ((), ())),
            preferred_element_type=jnp.float32)
        h = 0.5 * h * (1.0 + lax.erf(h * 0.7071067811865476))
        o_ref[...] = lax.dot_general(
            h, w2_ref[0], (((1,), (0,)), ((), ())),
            preferred_element_type=jnp.float32)


def _dispatch_body(N, P, xf_hbm, dst_hbm, xs_hbm,
                   idx0_v, idx1_v, rows_v, gsems, ssems):
    wid = lax.axis_index("s") * NC + lax.axis_index("c")
    TPW = N // NW
    CH = 16
    NCH = TPW // CH
    DEPTH = len(rows_v)
    t0 = wid * TPW
    pltpu.sync_copy(dst_hbm.at[pl.ds(t0, TPW)], idx0_v)
    pltpu.sync_copy(dst_hbm.at[pl.ds(N + t0, TPW)], idx1_v)
    for c in range(min(DEPTH, NCH)):
        pltpu.async_copy(xf_hbm.at[pl.ds(t0 + c * CH, CH)], rows_v[c],
                         gsems[c])
    for c in range(NCH):
        b = c % DEPTH
        iv0 = idx0_v[pl.ds(c * CH, CH)]
        iv1 = idx1_v[pl.ds(c * CH, CH)]
        pltpu.make_async_copy(xf_hbm.at[pl.ds(t0 + c * CH, CH)], rows_v[b],
                              gsems[b]).wait()
        s0 = pltpu.async_copy(rows_v[b], xs_hbm.at[iv0], ssems[b])
        s1 = pltpu.async_copy(rows_v[b], xs_hbm.at[iv1], ssems[b])
        if c + DEPTH < NCH:
            s0.wait()
            s1.wait()
            pltpu.async_copy(xf_hbm.at[pl.ds(t0 + (c + DEPTH) * CH, CH)],
                             rows_v[b], gsems[b])
    for c in range(max(NCH - DEPTH, 0), NCH):
        b = c % DEPTH
        iv0 = idx0_v[pl.ds(c * CH, CH)]
        iv1 = idx1_v[pl.ds(c * CH, CH)]
        pltpu.make_async_copy(rows_v[b], xs_hbm.at[iv0], ssems[b]).wait()
        pltpu.make_async_copy(rows_v[b], xs_hbm.at[iv1], ssems[b]).wait()


def _bcast_lane(vec, i):
    idx = jnp.full((LANES, 1), i, jnp.int32)
    dn = lax.GatherDimensionNumbers(
        offset_dims=(), collapsed_slice_dims=(0,), start_index_map=(0,))
    return lax.gather(vec, idx, dn, (1,),
                      mode=lax.GatherScatterMode.PROMISE_IN_BOUNDS)


def _combine_body(N, P, y_hbm, dst_hbm, wb_hbm, out_hbm,
                  i0_v, i1_v, w0_v, w1_v, r0_v, r1_v, o_v,
                  g0sems, g1sems, osems):
    wid = lax.axis_index("s") * NC + lax.axis_index("c")
    TPW = N // NW
    CH = 16
    NCH = TPW // CH
    DEPTH = len(o_v)
    t0 = wid * TPW
    pltpu.sync_copy(dst_hbm.at[pl.ds(t0, TPW)], i0_v)
    pltpu.sync_copy(dst_hbm.at[pl.ds(N + t0, TPW)], i1_v)
    pltpu.sync_copy(wb_hbm.at[pl.ds(t0, TPW)], w0_v)
    pltpu.sync_copy(wb_hbm.at[pl.ds(N + t0, TPW)], w1_v)
    for c in range(min(DEPTH, NCH)):
        pltpu.async_copy(y_hbm.at[i0_v[pl.ds(c * CH, CH)]], r0_v[c],
                         g0sems[c])
        pltpu.async_copy(y_hbm.at[i1_v[pl.ds(c * CH, CH)]], r1_v[c],
                         g1sems[c])
    for c in range(NCH):
        b = c % DEPTH
        pltpu.make_async_copy(y_hbm.at[i0_v[pl.ds(c * CH, CH)]], r0_v[b],
                              g0sems[b]).wait()
        pltpu.make_async_copy(y_hbm.at[i1_v[pl.ds(c * CH, CH)]], r1_v[b],
                              g1sems[b]).wait()
        if c >= DEPTH:
            pltpu.make_async_copy(
                o_v[b], out_hbm.at[pl.ds(t0 + (c - DEPTH) * CH, CH)],
                osems[b]).wait()

        wv0 = w0_v[pl.ds(c * CH, CH)]
        wv1 = w1_v[pl.ds(c * CH, CH)]

        def row(i, _):
            wa = _bcast_lane(wv0, i)
            wb = _bcast_lane(wv1, i)
            def col(j, _):
                a = r0_v[b][i, pl.ds(j * LANES, LANES)]
                d = r1_v[b][i, pl.ds(j * LANES, LANES)]
                o_v[b][i, pl.ds(j * LANES, LANES)] = wa * a + wb * d
                return 0
            return lax.fori_loop(0, DIM // LANES, col, 0)
        lax.fori_loop(0, CH, row, 0)
        pltpu.async_copy(o_v[b], out_hbm.at[pl.ds(t0 + c * CH, CH)], osems[b])
        if c + DEPTH < NCH:
            pltpu.async_copy(y_hbm.at[i0_v[pl.ds((c + DEPTH) * CH, CH)]],
                             r0_v[b], g0sems[b])
            pltpu.async_copy(y_hbm.at[i1_v[pl.ds((c + DEPTH) * CH, CH)]],
                             r1_v[b], g1sems[b])
    for c in range(max(NCH - DEPTH, 0), NCH):
        b = c % DEPTH
        pltpu.make_async_copy(o_v[b], out_hbm.at[pl.ds(t0 + c * CH, CH)],
                              osems[b]).wait()


def kernel(x, W_router, W1, W2):
    B, T, _ = x.shape
    N = B * T
    P = N * K
    NB = P // BLK + E
    PAD = NB * BLK
    xf = x.reshape(N, DIM)

    dst2, w2d, be2 = pl.pallas_call(
        functools.partial(_router_body, N, P, NB),
        out_shape=(
            jax.ShapeDtypeStruct((1, P), jnp.int32),
            jax.ShapeDtypeStruct((1, P), jnp.float32),
            jax.ShapeDtypeStruct((1, NB + 1), jnp.int32),
        ),
    )(xf, W_router)
    dst = dst2.reshape(P)
    w_flat = w2d.reshape(P)
    block_expert = be2.reshape(NB + 1)

    mesh = plsc.VectorSubcoreMesh(core_axis_name="c", subcore_axis_name="s",
                                  num_cores=NC, num_subcores=NS)
    x_sorted = pl.kernel(
        functools.partial(_dispatch_body, N, P),
        out_type=jax.ShapeDtypeStruct((PAD, DIM), jnp.float32),
        mesh=mesh,
        scratch_types=[
            pltpu.VMEM((N // NW,), jnp.int32),
            pltpu.VMEM((N // NW,), jnp.int32),
            (pltpu.VMEM((16, DIM), jnp.float32),) * 3,
            (pltpu.SemaphoreType.DMA,) * 3,
            (pltpu.SemaphoreType.DMA,) * 3,
        ],
    )(xf, dst)

    y_sorted = pl.pallas_call(
        functools.partial(_ffn_body, NB),
        grid_spec=pltpu.PrefetchScalarGridSpec(
            num_scalar_prefetch=1,
            grid=(NB,),
            in_specs=[
                pl.BlockSpec((BLK, DIM), lambda b, be: (b, 0)),
                pl.BlockSpec((1, DIM, FF), lambda b, be: (be[b], 0, 0),
                             pipeline_mode=pl.Buffered(buffer_count=2)),
                pl.BlockSpec((1, FF, DIM), lambda b, be: (be[b], 0, 0),
                             pipeline_mode=pl.Buffered(buffer_count=1)),
            ],
            out_specs=pl.BlockSpec((BLK, DIM), lambda b, be: (b, 0)),
        ),
        out_shape=jax.ShapeDtypeStruct((PAD, DIM), jnp.float32),
    )(block_expert, x_sorted, W1, W2)

    TPW = N // NW
    out = pl.kernel(
        functools.partial(_combine_body, N, P),
        out_type=jax.ShapeDtypeStruct((N, DIM), jnp.float32),
        mesh=mesh,
        scratch_types=[
            pltpu.VMEM((TPW,), jnp.int32),
            pltpu.VMEM((TPW,), jnp.int32),
            pltpu.VMEM((TPW,), jnp.float32),
            pltpu.VMEM((TPW,), jnp.float32),
            (pltpu.VMEM((16, DIM), jnp.float32),) * 2,
            (pltpu.VMEM((16, DIM), jnp.float32),) * 2,
            (pltpu.VMEM((16, DIM), jnp.float32),) * 2,
            (pltpu.SemaphoreType.DMA,) * 2,
            (pltpu.SemaphoreType.DMA,) * 2,
            (pltpu.SemaphoreType.DMA,) * 2,
        ],
    )(y_sorted, dst, w_flat)

    return out.reshape(B, T, DIM)
